# baseline (speedup 1.0000x reference)
import jax
import jax.numpy as jnp
from jax.experimental import pallas as pl

_NC = 25
_B = 1024
_S = 1024


def _onehot_body(idx_ref, out_ref):
    idx = idx_ref[...]
    classes = jax.lax.broadcasted_iota(jnp.int32, (_NC, 1, 1), 0)
    out_ref[...] = (idx[None, :, :] == classes).astype(jnp.float32)


def kernel(inputs):
    bm, bs = 8, 1024
    grid = (_B // bm, _S // bs)
    y = pl.pallas_call(
        _onehot_body,
        grid=grid,
        in_specs=[pl.BlockSpec((bm, bs), lambda i, j: (i, j))],
        out_specs=pl.BlockSpec((_NC, bm, bs), lambda i, j: (0, i, j)),
        out_shape=jax.ShapeDtypeStruct((_NC, _B, _S), jnp.float32),
    )(inputs)
    return jnp.transpose(y, (1, 2, 0))


# speedup vs baseline: 2.4927x; 2.4927x over previous
import jax
import jax.numpy as jnp
from jax.experimental import pallas as pl

_NC = 25
_B = 1024
_S = 1024


def _onehot_body(idx_ref, out_ref):
    c = pl.program_id(0)
    idx = idx_ref[...]
    out_ref[0] = (idx == c).astype(jnp.float32)


def kernel(inputs):
    y = pl.pallas_call(
        _onehot_body,
        grid=(_NC,),
        in_specs=[pl.BlockSpec((_B, _S), lambda c: (0, 0))],
        out_specs=pl.BlockSpec((1, _B, _S), lambda c: (c, 0, 0)),
        out_shape=jax.ShapeDtypeStruct((_NC, _B, _S), jnp.float32),
    )(inputs)
    return jnp.transpose(y, (1, 2, 0))
